# Initial kernel scaffold; baseline (speedup 1.0000x reference)
import jax
import jax.numpy as jnp
from jax.experimental import pallas as pl


def kernel(x, edge_index, batch, W0, b0, g0, be0, rm0, rv0, W1, b1, g1, be1, rm1, rv1, W2, b2, g2, be2, rm2, rv2, Wp1, bp1, Wp2, bp2, Wa1, ba1, Wa2, ba2, Wt1, bt1, Wt2, bt2):
    raise NotImplementedError("write your pallas kernel here")


# speedup vs baseline: 2.3166x; 2.3166x over previous
import functools
import jax
import jax.numpy as jnp
from jax.experimental import pallas as pl

N = 100000
E = 1600000
H = 64
EPS = 1e-5
BN_ROWS = 2000


def _mm_body(prev_ref, w_ref, dis_ref, out_ref):
    out_ref[...] = (prev_ref[...] @ w_ref[...]) * dis_ref[...]


def _matmul_u(prev, W, dis):
    n, k = prev.shape
    h = W.shape[1]
    grid = (n // BN_ROWS,)
    return pl.pallas_call(
        _mm_body,
        grid=grid,
        in_specs=[
            pl.BlockSpec((BN_ROWS, k), lambda i: (i, 0)),
            pl.BlockSpec((k, h), lambda i: (0, 0)),
            pl.BlockSpec((BN_ROWS, 1), lambda i: (i, 0)),
        ],
        out_specs=pl.BlockSpec((BN_ROWS, h), lambda i: (i, 0)),
        out_shape=jax.ShapeDtypeStruct((n, h), jnp.float32),
    )(prev, W, dis)


def kernel(x, edge_index, batch, W0, b0, g0, be0, rm0, rv0, W1, b1, g1, be1,
           rm1, rv1, W2, b2, g2, be2, rm2, rv2, Wp1, bp1, Wp2, bp2, Wa1, ba1,
           Wa2, ba2, Wt1, bt1, Wt2, bt2):
    src = edge_index[0]
    dst = edge_index[1]
    deg = jnp.ones((N,), jnp.float32).at[dst].add(1.0)
    dis = (1.0 / jnp.sqrt(deg))[:, None]

    h = x
    params = [(W0, b0, g0, be0, rm0, rv0), (W1, b1, g1, be1, rm1, rv1),
              (W2, b2, g2, be2, rm2, rv2)]
    for (W, b, g, be, rm, rv) in params:
        u = _matmul_u(h, W, dis)
        acc = jnp.zeros((N, H), jnp.float32).at[dst].add(u[src])
        q = g / jnp.sqrt(rv + EPS)
        s = q
        t = (b - rm) * q + be
        h = jax.nn.relu(dis * (acc + u) * s + t)

    pooled = jnp.mean(h, axis=0, keepdims=True)
    power = (jax.nn.relu(pooled @ Wp1 + bp1) @ Wp2 + bp2)[:, 0]
    area = (jax.nn.relu(pooled @ Wa1 + ba1) @ Wa2 + ba2)[:, 0]
    timing = jax.nn.relu(pooled @ Wt1 + bt1) @ Wt2 + bt2
    return power, area, timing[:, 0], timing[:, 1]


# speedup vs baseline: 10.3205x; 4.4549x over previous
import functools
import jax
import jax.numpy as jnp
from jax import lax
from jax.experimental import pallas as pl
from jax.experimental.pallas import tpu as pltpu
from jax.experimental.pallas import tpu_sc as plsc

N = 100000
E = 1600000
H = 64
EPS = 1e-5

BN = 2000
NBLK = N // BN
NPAD_U = N + BN
NOPAD = 100096
ACC_ROWS = NOPAD
ZSTRIPE = ACC_ROWS // 16
OSTRIPE = NOPAD // 16
EROWS = 12544
EPAD = EROWS * 128
TROWS = EROWS // 16
TCHUNKS = TROWS // 8
DROWS = EROWS // 2 // 16
DCHUNKS = DROWS // 8

_mesh = plsc.VectorSubcoreMesh(core_axis_name="c", subcore_axis_name="s",
                               num_cores=2, num_subcores=16)


@functools.partial(
    pl.kernel,
    out_type=jax.ShapeDtypeStruct((2, NOPAD, 16), jnp.float32),
    mesh=_mesh,
    scratch_types=[
        pltpu.VMEM((8, 128), jnp.int32),
        pltpu.VMEM((128, 16), jnp.float32),
        pltpu.VMEM_SHARED((ACC_ROWS, 16), jnp.float32),
    ],
    compiler_params=pltpu.CompilerParams(use_tc_tiling_on_sc=False),
)
def _deg_sc(dst_hbm, ones_hbm, zeros_hbm, out_hbm, dst_v, ones_v, acc_sh):
    c = lax.axis_index("c")
    s = lax.axis_index("s")
    pltpu.sync_copy(ones_hbm, ones_v)
    pltpu.sync_copy(zeros_hbm, acc_sh.at[pl.ds(s * ZSTRIPE, ZSTRIPE)])
    plsc.subcore_barrier()
    base = c * (EROWS // 2) + s * DROWS

    def chunk(t, carry):
        row0 = base + t * 8
        pltpu.sync_copy(dst_hbm.at[pl.ds(row0, 8)], dst_v)
        for jj in range(8):
            pltpu.sync_copy(ones_v, acc_sh.at[dst_v.at[jj]], add=True)
        return carry

    lax.fori_loop(0, DCHUNKS, chunk, 0)
    plsc.subcore_barrier()
    pltpu.sync_copy(acc_sh.at[pl.ds(s * OSTRIPE, OSTRIPE)],
                    out_hbm.at[c].at[pl.ds(s * OSTRIPE, OSTRIPE)])


@functools.partial(
    pl.kernel,
    out_type=jax.ShapeDtypeStruct((4, NOPAD, 16), jnp.float32),
    mesh=_mesh,
    scratch_types=[
        pltpu.VMEM((8, 128), jnp.int32),
        pltpu.VMEM((8, 128), jnp.int32),
        pltpu.VMEM((1024, 16), jnp.float32),
        pltpu.VMEM_SHARED((ACC_ROWS, 16), jnp.float32),
        pltpu.SemaphoreType.DMA,
    ],
    compiler_params=pltpu.CompilerParams(use_tc_tiling_on_sc=False),
)
def _prop_sc(u_hbm, src_hbm, dst_hbm, zeros_hbm, out_hbm,
             src_v, dst_v, rows_v, acc_sh, sem):
    c = lax.axis_index("c")
    s = lax.axis_index("s")
    for j in (0, 1):
        slab = c * 2 + j
        pltpu.sync_copy(zeros_hbm, acc_sh.at[pl.ds(s * ZSTRIPE, ZSTRIPE)])
        plsc.subcore_barrier()

        def chunk(t, carry):
            row0 = s * TROWS + t * 8
            pltpu.sync_copy(src_hbm.at[pl.ds(row0, 8)], src_v)
            pltpu.sync_copy(dst_hbm.at[pl.ds(row0, 8)], dst_v)
            descs = [
                pltpu.async_copy(u_hbm.at[slab].at[src_v.at[jj]],
                                 rows_v.at[pl.ds(jj * 128, 128)], sem)
                for jj in range(8)
            ]
            for d in descs:
                d.wait()
            for jj in range(8):
                pltpu.sync_copy(rows_v.at[pl.ds(jj * 128, 128)],
                                acc_sh.at[dst_v.at[jj]], add=True)
            return carry

        lax.fori_loop(0, TCHUNKS, chunk, 0)
        plsc.subcore_barrier()
        pltpu.sync_copy(acc_sh.at[pl.ds(s * OSTRIPE, OSTRIPE)],
                        out_hbm.at[slab].at[pl.ds(s * OSTRIPE, OSTRIPE)])
        plsc.subcore_barrier()


def _dot(a, b, precision=jax.lax.Precision.DEFAULT):
    return jax.lax.dot_general(a, b, (((1,), (0,)), ((), ())),
                               precision=precision,
                               preferred_element_type=jnp.float32)


def _mm_body(x_ref, w_ref, deg_ref, u_ref):
    dis = 1.0 / jnp.sqrt(deg_ref[0] + deg_ref[1] + 1.0)
    u_ref[...] = (_dot(x_ref[...], w_ref[0]) * dis)[None]


def _mm(x, W, deg):
    kdim = x.shape[1]
    Wr = W.reshape(kdim, 4, 16).transpose(1, 0, 2)
    return pl.pallas_call(
        _mm_body,
        grid=(4, NBLK),
        in_specs=[
            pl.BlockSpec((BN, kdim), lambda j, i: (i, 0)),
            pl.BlockSpec((1, kdim, 16), lambda j, i: (j, 0, 0)),
            pl.BlockSpec((2, BN, 16), lambda j, i: (0, i, 0)),
        ],
        out_specs=pl.BlockSpec((1, BN, 16), lambda j, i: (j, i, 0)),
        out_shape=jax.ShapeDtypeStruct((4, NPAD_U, 16), jnp.float32),
    )(x, Wr, deg)


def _fuse_body(acc_ref, u_ref, deg_ref, s_ref, t_ref, y_ref):
    dis = 1.0 / jnp.sqrt(deg_ref[0] + deg_ref[1] + 1.0)
    ys = []
    for k in range(4):
        y = dis * (acc_ref[k] + u_ref[k]) * s_ref[k] + t_ref[k]
        ys.append(jnp.maximum(y, 0.0))
    y_ref[...] = jnp.concatenate(ys, axis=1)


def _fuse(acc, u, deg, svec, tvec):
    return pl.pallas_call(
        _fuse_body,
        grid=(NBLK,),
        in_specs=[
            pl.BlockSpec((4, BN, 16), lambda i: (0, i, 0)),
            pl.BlockSpec((4, BN, 16), lambda i: (0, i, 0)),
            pl.BlockSpec((2, BN, 16), lambda i: (0, i, 0)),
            pl.BlockSpec((4, 1, 16), lambda i: (0, 0, 0)),
            pl.BlockSpec((4, 1, 16), lambda i: (0, 0, 0)),
        ],
        out_specs=pl.BlockSpec((BN, H), lambda i: (i, 0)),
        out_shape=jax.ShapeDtypeStruct((N, H), jnp.float32),
    )(acc, u, deg, svec, tvec)


def _final_body(acc_ref, u_ref, deg_ref, s_ref, t_ref,
                wp1_ref, bp1_ref, wp2_ref, bp2_ref,
                wa1_ref, ba1_ref, wa2_ref, ba2_ref,
                wt1_ref, bt1_ref, wt2_ref, bt2_ref,
                out_ref, acc_scr):
    i = pl.program_id(0)

    @pl.when(i == 0)
    def _():
        acc_scr[...] = jnp.zeros((4, 16), jnp.float32)

    dis = 1.0 / jnp.sqrt(deg_ref[0] + deg_ref[1] + 1.0)
    parts = []
    for k in range(4):
        y = dis * (acc_ref[k] + u_ref[k]) * s_ref[k] + t_ref[k]
        y = jnp.maximum(y, 0.0)
        parts.append(jnp.sum(y, axis=0, keepdims=True))
    acc_scr[...] += jnp.concatenate(parts, axis=0)

    @pl.when(i == NBLK - 1)
    def _():
        pooled = acc_scr[...] * (1.0 / N)

        def head(w1_ref, b1_ref, w2_ref, b2_ref):
            z = jnp.zeros((1, w1_ref.shape[1]), jnp.float32)
            for k in range(4):
                z = z + pooled[k:k + 1, :] @ w1_ref[16 * k:16 * (k + 1), :]
            z = jnp.maximum(z + b1_ref[...], 0.0)
            return z @ w2_ref[...] + b2_ref[...]

        p = head(wp1_ref, bp1_ref, wp2_ref, bp2_ref)
        a = head(wa1_ref, ba1_ref, wa2_ref, ba2_ref)
        t = head(wt1_ref, bt1_ref, wt2_ref, bt2_ref)
        out_ref[...] = jnp.concatenate(
            [p, a, t, jnp.zeros((1, 124), jnp.float32)], axis=1)


def _final(acc, u, deg, svec, tvec, Wp1, bp1, Wp2, bp2,
           Wa1, ba1, Wa2, ba2, Wt1, bt1, Wt2, bt2):
    full = lambda i: (0, 0)
    full3 = lambda i: (0, 0, 0)
    return pl.pallas_call(
        _final_body,
        grid=(NBLK,),
        in_specs=[
            pl.BlockSpec((4, BN, 16), lambda i: (0, i, 0)),
            pl.BlockSpec((4, BN, 16), lambda i: (0, i, 0)),
            pl.BlockSpec((2, BN, 16), lambda i: (0, i, 0)),
            pl.BlockSpec((4, 1, 16), full3),
            pl.BlockSpec((4, 1, 16), full3),
            pl.BlockSpec((H, 32), full),
            pl.BlockSpec((1, 32), full),
            pl.BlockSpec((32, 1), full),
            pl.BlockSpec((1, 1), full),
            pl.BlockSpec((H, 32), full),
            pl.BlockSpec((1, 32), full),
            pl.BlockSpec((32, 1), full),
            pl.BlockSpec((1, 1), full),
            pl.BlockSpec((H, 32), full),
            pl.BlockSpec((1, 32), full),
            pl.BlockSpec((32, 2), full),
            pl.BlockSpec((1, 2), full),
        ],
        out_specs=pl.BlockSpec((1, 128), full),
        out_shape=jax.ShapeDtypeStruct((1, 128), jnp.float32),
        scratch_shapes=[pltpu.VMEM((4, 16), jnp.float32)],
    )(acc, u, deg, svec, tvec, Wp1, bp1, Wp2, bp2,
      Wa1, ba1, Wa2, ba2, Wt1, bt1, Wt2, bt2)


def _fold_bn(b, g, be, rm, rv):
    q = g / jnp.sqrt(rv + EPS)
    return q.reshape(4, 1, 16), ((b - rm) * q + be).reshape(4, 1, 16)


def kernel(x, edge_index, batch, W0, b0, g0, be0, rm0, rv0, W1, b1, g1, be1,
           rm1, rv1, W2, b2, g2, be2, rm2, rv2, Wp1, bp1, Wp2, bp2, Wa1, ba1,
           Wa2, ba2, Wt1, bt1, Wt2, bt2):
    dummy = jnp.full((EPAD - E,), N, jnp.int32)
    src2 = jnp.concatenate([edge_index[0], dummy]).reshape(EROWS, 128)
    dst2 = jnp.concatenate([edge_index[1], dummy]).reshape(EROWS, 128)
    ones16 = jnp.ones((128, 16), jnp.float32)
    zeros16 = jnp.zeros((ZSTRIPE, 16), jnp.float32)

    deg = _deg_sc(dst2, ones16, zeros16)

    s0, t0 = _fold_bn(b0, g0, be0, rm0, rv0)
    s1, t1 = _fold_bn(b1, g1, be1, rm1, rv1)
    s2, t2 = _fold_bn(b2, g2, be2, rm2, rv2)

    u = _mm(x, W0, deg)
    acc = _prop_sc(u, src2, dst2, zeros16)
    y = _fuse(acc, u, deg, s0, t0)
    u = _mm(y, W1, deg)
    acc = _prop_sc(u, src2, dst2, zeros16)
    y = _fuse(acc, u, deg, s1, t1)
    u = _mm(y, W2, deg)
    acc = _prop_sc(u, src2, dst2, zeros16)

    res = _final(acc, u, deg, s2, t2,
                 Wp1, bp1.reshape(1, 32), Wp2, bp2.reshape(1, 1),
                 Wa1, ba1.reshape(1, 32), Wa2, ba2.reshape(1, 1),
                 Wt1, bt1.reshape(1, 32), Wt2, bt2.reshape(1, 2))[0]
    return res[0:1], res[1:2], res[2:3], res[3:4]
